# Initial kernel scaffold; baseline (speedup 1.0000x reference)
import jax
import jax.numpy as jnp
from jax.experimental import pallas as pl


def kernel(power, time_step, edge_index):
    raise NotImplementedError("write your pallas kernel here")


# speedup vs baseline: 251.5855x; 251.5855x over previous
import functools

import jax
import jax.numpy as jnp
from jax import lax
from jax.experimental import pallas as pl
from jax.experimental.pallas import tpu as pltpu
from jax.experimental.pallas import tpu_sc as plsc

N_NODES = 100000
N_EDGES = 6400000
NC = 2
NS = 16
NW = NC * NS
EPW = N_EDGES // NW
CHUNK = 4000
NCHUNKS = EPW // CHUNK
NPAD = 100352
SLICE = NPAD // NS

_mesh = plsc.VectorSubcoreMesh(core_axis_name="c", subcore_axis_name="s")


@functools.partial(
    pl.kernel,
    out_type=jax.ShapeDtypeStruct((NC, NPAD), jnp.float32),
    mesh=_mesh,
    scratch_types=[
        pltpu.VMEM_SHARED((N_NODES,), jnp.float32),
        pltpu.VMEM_SHARED((NPAD,), jnp.float32),
        pltpu.VMEM((CHUNK,), jnp.int32),
        pltpu.VMEM((CHUNK,), jnp.int32),
        pltpu.VMEM((CHUNK,), jnp.float32),
        pltpu.VMEM((SLICE,), jnp.float32),
        pltpu.SemaphoreType.DMA,
    ],
)
def _edge_pool_kernel(power_hbm, edge_hbm, part_hbm,
                      table, acc, src_v, dst_v, vals_v, buf_v, sem):
    ci = lax.axis_index("c")
    si = lax.axis_index("s")
    wid = ci * NS + si

    def zero_body(i, carry):
        buf_v[pl.ds(pl.multiple_of(i * 16, 16), 16)] = jnp.zeros((16,), jnp.float32)
        return carry
    lax.fori_loop(0, SLICE // 16, zero_body, 0)
    acc_off = pl.multiple_of(si * SLICE, 8)
    pltpu.sync_copy(buf_v, acc.at[pl.ds(acc_off, SLICE)])

    @pl.when(si == 0)
    def _():
        pltpu.sync_copy(power_hbm, table)

    plsc.subcore_barrier()

    def chunk_body(k, carry):
        base = pl.multiple_of(wid * EPW + k * CHUNK, 8)
        pltpu.sync_copy(edge_hbm.at[pl.ds(base, CHUNK)], src_v)
        pltpu.sync_copy(edge_hbm.at[pl.ds(N_EDGES + base, CHUNK)], dst_v)
        pltpu.async_copy(table.at[src_v], vals_v, sem).wait()
        pltpu.sync_copy(vals_v, acc.at[dst_v], add=True)
        return carry
    lax.fori_loop(0, NCHUNKS, chunk_body, 0)

    plsc.subcore_barrier()

    pltpu.sync_copy(acc.at[pl.ds(acc_off, SLICE)], buf_v)
    pltpu.sync_copy(buf_v, part_hbm.at[ci, pl.ds(acc_off, SLICE)])


def _combine_body(ts_ref, p_ref, o_ref):
    o_ref[...] = (p_ref[0] + p_ref[1]) * ts_ref[0]


_combine = pl.pallas_call(
    _combine_body,
    out_shape=jax.ShapeDtypeStruct((NPAD // 128, 128), jnp.float32),
    in_specs=[
        pl.BlockSpec(memory_space=pltpu.SMEM),
        pl.BlockSpec(memory_space=pltpu.VMEM),
    ],
    out_specs=pl.BlockSpec(memory_space=pltpu.VMEM),
)


@jax.jit
def kernel(power, time_step, edge_index):
    edges = edge_index.astype(jnp.int32).reshape(-1)
    partials = _edge_pool_kernel(power, edges)
    heat_pad = _combine(time_step, partials.reshape(NC, NPAD // 128, 128))
    return heat_pad.reshape(-1)[:N_NODES]
